# baseline (speedup 1.0000x reference)
import functools

import jax
import jax.numpy as jnp
from jax.experimental import pallas as pl
from jax.experimental.pallas import tpu as pltpu

B, N, C_IN = 4, 100000, 4
FEAT = 128
H, W = 256, 256
PCR = (-50.0, -50.0, -5.0, 50.0, 50.0, 3.0)
EPS = 1e-5

P_CHUNK = 4000
N_CHUNKS = (B * N) // P_CHUNK
CHUNKS_PER_BATCH = N // P_CHUNK


def _mlp_body(x_ref, w1_ref, b1_ref, w2_ref, b2_ref, w3_ref, b3_ref,
              feats_ref, lin_ref):
    x = x_ref[...]
    h = jnp.maximum(jnp.dot(x, w1_ref[...], preferred_element_type=jnp.float32) + b1_ref[...], 0.0)
    h = jnp.maximum(jnp.dot(h, w2_ref[...], preferred_element_type=jnp.float32) + b2_ref[...], 0.0)
    h = jnp.maximum(jnp.dot(h, w3_ref[...], preferred_element_type=jnp.float32) + b3_ref[...], 0.0)

    xn = (x[:, 0:1] - PCR[0]) / (PCR[3] - PCR[0])
    yn = (x[:, 1:2] - PCR[1]) / (PCR[4] - PCR[1])
    valid = (xn >= 0.0) & (xn <= 1.0) & (yn >= 0.0) & (yn <= 1.0)
    gx = jnp.clip((xn * (W - 1)).astype(jnp.int32), 0, W - 1)
    gy = jnp.clip((yn * (H - 1)).astype(jnp.int32), 0, H - 1)
    b = pl.program_id(0) // CHUNKS_PER_BATCH
    lin = b * (H * W) + gy * W + gx

    feats_ref[...] = jnp.where(valid, h, -jnp.inf)
    lin_ref[...] = lin


@functools.partial(jax.jit, static_argnums=())
def _mlp_pallas(pts_flat, w1t, b1r, w2t, b2r, w3t, b3r):
    grid = (N_CHUNKS,)
    return pl.pallas_call(
        _mlp_body,
        grid=grid,
        in_specs=[
            pl.BlockSpec((P_CHUNK, C_IN), lambda i: (i, 0)),
            pl.BlockSpec((C_IN, 64), lambda i: (0, 0)),
            pl.BlockSpec((1, 64), lambda i: (0, 0)),
            pl.BlockSpec((64, 128), lambda i: (0, 0)),
            pl.BlockSpec((1, 128), lambda i: (0, 0)),
            pl.BlockSpec((128, FEAT), lambda i: (0, 0)),
            pl.BlockSpec((1, FEAT), lambda i: (0, 0)),
        ],
        out_specs=[
            pl.BlockSpec((P_CHUNK, FEAT), lambda i: (i, 0)),
            pl.BlockSpec((P_CHUNK, 1), lambda i: (i, 0)),
        ],
        out_shape=[
            jax.ShapeDtypeStruct((B * N, FEAT), jnp.float32),
            jax.ShapeDtypeStruct((B * N, 1), jnp.int32),
        ],
    )(pts_flat, w1t, b1r, w2t, b2r, w3t, b3r)


def kernel(points, w1, b1, g1, be1, m1, v1, w2, b2, g2, be2, m2, v2, w3, b3, g3, be3, m3, v3):
    def fold(wt, bb, g, be, m, v):
        s = g * jax.lax.rsqrt(v + EPS)
        return (wt.T * s[None, :]), (s * (bb - m) + be)

    w1t, b1r = fold(w1, b1, g1, be1, m1, v1)
    w2t, b2r = fold(w2, b2, g2, be2, m2, v2)
    w3t, b3r = fold(w3, b3, g3, be3, m3, v3)

    pts_flat = points.reshape(B * N, C_IN)
    feats, lin = _mlp_pallas(pts_flat, w1t, b1r[None, :], w2t, b2r[None, :], w3t, b3r[None, :])

    lin_flat = lin.reshape(-1)
    grid = jnp.full((B * H * W, FEAT), -jnp.inf, dtype=jnp.float32).at[lin_flat].max(feats)
    grid = jnp.where(jnp.isneginf(grid), 0.0, grid)
    return grid.reshape(B, H, W, FEAT).transpose(0, 3, 1, 2)


# speedup vs baseline: 1.0567x; 1.0567x over previous
import functools

import jax
import jax.numpy as jnp
from jax.experimental import pallas as pl
from jax.experimental.pallas import tpu as pltpu

B, N, C_IN = 4, 100000, 4
FEAT = 128
H, W = 256, 256
PCR = (-50.0, -50.0, -5.0, 50.0, 50.0, 3.0)
EPS = 1e-5

P_CHUNK = 4000
N_CHUNKS = (B * N) // P_CHUNK
CHUNKS_PER_BATCH = N // P_CHUNK


def _mlp_body(x_ref, xt_ref, w1_ref, b1_ref, w2_ref, b2_ref, w3_ref, b3_ref,
              feats_ref, lin_ref):
    x = x_ref[...]
    h = jnp.maximum(jnp.dot(x, w1_ref[...], preferred_element_type=jnp.float32) + b1_ref[...], 0.0)
    h = jnp.maximum(jnp.dot(h, w2_ref[...], preferred_element_type=jnp.float32) + b2_ref[...], 0.0)
    h = jnp.maximum(jnp.dot(h, w3_ref[...], preferred_element_type=jnp.float32) + b3_ref[...], 0.0)

    xc = x[:, 0:1]
    yc = x[:, 1:2]
    xn_c = (xc - PCR[0]) / (PCR[3] - PCR[0])
    yn_c = (yc - PCR[1]) / (PCR[4] - PCR[1])
    valid_c = (xn_c >= 0.0) & (xn_c <= 1.0) & (yn_c >= 0.0) & (yn_c <= 1.0)
    feats_ref[...] = jnp.where(valid_c, h, -jnp.inf)

    xr = xt_ref[0, 0:1, :]
    yr = xt_ref[0, 1:2, :]
    xn = (xr - PCR[0]) / (PCR[3] - PCR[0])
    yn = (yr - PCR[1]) / (PCR[4] - PCR[1])
    gx = jnp.clip((xn * (W - 1)).astype(jnp.int32), 0, W - 1)
    gy = jnp.clip((yn * (H - 1)).astype(jnp.int32), 0, H - 1)
    b = pl.program_id(0) // CHUNKS_PER_BATCH
    lin_ref[...] = (b * (H * W) + gy * W + gx)[None]


@jax.jit
def _mlp_pallas(pts_flat, pts_t, w1t, b1r, w2t, b2r, w3t, b3r):
    grid = (N_CHUNKS,)
    return pl.pallas_call(
        _mlp_body,
        grid=grid,
        in_specs=[
            pl.BlockSpec((P_CHUNK, C_IN), lambda i: (i, 0)),
            pl.BlockSpec((1, C_IN, P_CHUNK), lambda i: (i, 0, 0)),
            pl.BlockSpec((C_IN, 64), lambda i: (0, 0)),
            pl.BlockSpec((1, 64), lambda i: (0, 0)),
            pl.BlockSpec((64, 128), lambda i: (0, 0)),
            pl.BlockSpec((1, 128), lambda i: (0, 0)),
            pl.BlockSpec((128, FEAT), lambda i: (0, 0)),
            pl.BlockSpec((1, FEAT), lambda i: (0, 0)),
        ],
        out_specs=[
            pl.BlockSpec((P_CHUNK, FEAT), lambda i: (i, 0)),
            pl.BlockSpec((1, 1, P_CHUNK), lambda i: (i, 0, 0)),
        ],
        out_shape=[
            jax.ShapeDtypeStruct((B * N, FEAT), jnp.float32),
            jax.ShapeDtypeStruct((N_CHUNKS, 1, P_CHUNK), jnp.int32),
        ],
    )(pts_flat, pts_t, w1t, b1r, w2t, b2r, w3t, b3r)


def kernel(points, w1, b1, g1, be1, m1, v1, w2, b2, g2, be2, m2, v2, w3, b3, g3, be3, m3, v3):
    def fold(wt, bb, g, be, m, v):
        s = g * jax.lax.rsqrt(v + EPS)
        return (wt.T * s[None, :]), (s * (bb - m) + be)

    w1t, b1r = fold(w1, b1, g1, be1, m1, v1)
    w2t, b2r = fold(w2, b2, g2, be2, m2, v2)
    w3t, b3r = fold(w3, b3, g3, be3, m3, v3)

    pts_flat = points.reshape(B * N, C_IN)
    pts_t = pts_flat.reshape(N_CHUNKS, P_CHUNK, C_IN).transpose(0, 2, 1)
    feats, lin = _mlp_pallas(pts_flat, pts_t, w1t, b1r[None, :], w2t, b2r[None, :],
                             w3t, b3r[None, :])

    lin_flat = lin.reshape(-1)
    grid = jnp.full((B * H * W, FEAT), -jnp.inf, dtype=jnp.float32).at[lin_flat].max(feats)
    grid = jnp.where(jnp.isneginf(grid), 0.0, grid)
    return grid.reshape(B, H, W, FEAT).transpose(0, 3, 1, 2)


# speedup vs baseline: 1.3177x; 1.2470x over previous
import functools

import jax
import jax.numpy as jnp
from jax.experimental import pallas as pl
from jax.experimental.pallas import tpu as pltpu

B, N, C_IN = 4, 100000, 4
FEAT = 128
H, W = 256, 256
PCR = (-50.0, -50.0, -5.0, 50.0, 50.0, 3.0)
EPS = 1e-5

P_CHUNK = 4000
N_CHUNKS = N // P_CHUNK


def _mlp_body(x_ref, xt_ref, w1_ref, b1_ref, w2_ref, b2_ref, w3_ref, b3_ref,
              feats_ref, lin_ref):
    x = x_ref[...]
    h = jnp.maximum(jnp.dot(x, w1_ref[...], preferred_element_type=jnp.float32) + b1_ref[...], 0.0)
    h = jnp.maximum(jnp.dot(h, w2_ref[...], preferred_element_type=jnp.float32) + b2_ref[...], 0.0)
    h = jnp.maximum(jnp.dot(h, w3_ref[...], preferred_element_type=jnp.float32) + b3_ref[...], 0.0)

    xc = x[:, 0:1]
    yc = x[:, 1:2]
    xn_c = (xc - PCR[0]) / (PCR[3] - PCR[0])
    yn_c = (yc - PCR[1]) / (PCR[4] - PCR[1])
    valid_c = (xn_c >= 0.0) & (xn_c <= 1.0) & (yn_c >= 0.0) & (yn_c <= 1.0)
    feats_ref[...] = jnp.where(valid_c, h, -jnp.inf)

    xr = xt_ref[0, 0:1, :]
    yr = xt_ref[0, 1:2, :]
    xn = (xr - PCR[0]) / (PCR[3] - PCR[0])
    yn = (yr - PCR[1]) / (PCR[4] - PCR[1])
    gx = jnp.clip((xn * (W - 1)).astype(jnp.int32), 0, W - 1)
    gy = jnp.clip((yn * (H - 1)).astype(jnp.int32), 0, H - 1)
    lin_ref[...] = (gy * W + gx)[None]


def _mlp_pallas(pts, pts_t, w1t, b1r, w2t, b2r, w3t, b3r):
    return pl.pallas_call(
        _mlp_body,
        grid=(N_CHUNKS,),
        in_specs=[
            pl.BlockSpec((P_CHUNK, C_IN), lambda i: (i, 0)),
            pl.BlockSpec((1, C_IN, P_CHUNK), lambda i: (i, 0, 0)),
            pl.BlockSpec((C_IN, 64), lambda i: (0, 0)),
            pl.BlockSpec((1, 64), lambda i: (0, 0)),
            pl.BlockSpec((64, 128), lambda i: (0, 0)),
            pl.BlockSpec((1, 128), lambda i: (0, 0)),
            pl.BlockSpec((128, FEAT), lambda i: (0, 0)),
            pl.BlockSpec((1, FEAT), lambda i: (0, 0)),
        ],
        out_specs=[
            pl.BlockSpec((P_CHUNK, FEAT), lambda i: (i, 0)),
            pl.BlockSpec((1, 1, P_CHUNK), lambda i: (i, 0, 0)),
        ],
        out_shape=[
            jax.ShapeDtypeStruct((N, FEAT), jnp.float32),
            jax.ShapeDtypeStruct((N_CHUNKS, 1, P_CHUNK), jnp.int32),
        ],
    )(pts, pts_t, w1t, b1r, w2t, b2r, w3t, b3r)


def kernel(points, w1, b1, g1, be1, m1, v1, w2, b2, g2, be2, m2, v2, w3, b3, g3, be3, m3, v3):
    def fold(wt, bb, g, be, m, v):
        s = g * jax.lax.rsqrt(v + EPS)
        return (wt.T * s[None, :]), (s * (bb - m) + be)

    w1t, b1r = fold(w1, b1, g1, be1, m1, v1)
    w2t, b2r = fold(w2, b2, g2, be2, m2, v2)
    w3t, b3r = fold(w3, b3, g3, be3, m3, v3)
    b1r, b2r, b3r = b1r[None, :], b2r[None, :], b3r[None, :]

    pts_t = points.reshape(B, N_CHUNKS, P_CHUNK, C_IN).transpose(0, 1, 3, 2)

    grids = []
    for b in range(B):
        feats, lin = _mlp_pallas(points[b], pts_t[b], w1t, b1r, w2t, b2r, w3t, b3r)
        g = jnp.full((H * W, FEAT), -jnp.inf, dtype=jnp.float32).at[lin.reshape(-1)].max(feats)
        grids.append(g)

    grid = jnp.stack(grids)
    grid = jnp.where(jnp.isneginf(grid), 0.0, grid)
    return grid.reshape(B, H, W, FEAT).transpose(0, 3, 1, 2)
